# baseline (speedup 1.0000x reference)
import jax
import jax.numpy as jnp
from jax import lax
from jax.experimental import pallas as pl

N_CODES = 1024
DIM = 64
TOKENS = 1024


def _vq_body(z_ref, e_ref, idx_ref, zq_ref):
    zb = z_ref[0]
    e = e_ref[...]
    eh = 0.5 * jnp.sum(e * e, axis=1, keepdims=True)
    e_aug = jnp.concatenate([e, eh], axis=1)
    z_aug = jnp.concatenate(
        [zb, jnp.full((1, TOKENS), -1.0, jnp.float32)], axis=0)
    s = lax.dot_general(
        e_aug, z_aug, (((1,), (0,)), ((), ())),
        preferred_element_type=jnp.float32)
    smax = jnp.max(s, axis=0, keepdims=True)
    iota = lax.broadcasted_iota(jnp.int32, (N_CODES, TOKENS), 0)
    masked = jnp.where(s == smax, iota, jnp.int32(N_CODES))
    idx = jnp.min(masked, axis=0)
    idx_ref[0, 0, :] = idx
    onehot = (masked == idx[None, :]).astype(jnp.bfloat16)
    zq = lax.dot_general(
        e.astype(jnp.bfloat16), onehot, (((0,), (0,)), ((), ())),
        preferred_element_type=jnp.float32)
    zq_ref[0] = zq


def kernel(z, embedding_weight):
    B, C, H, W = z.shape
    zf = z.reshape(B, C, H * W)
    grid = (B,)
    idx_out, zq_out = pl.pallas_call(
        _vq_body,
        grid=grid,
        in_specs=[
            pl.BlockSpec((1, C, H * W), lambda i: (i, 0, 0)),
            pl.BlockSpec((N_CODES, DIM), lambda i: (0, 0)),
        ],
        out_specs=[
            pl.BlockSpec((1, 1, H * W), lambda i: (i, 0, 0)),
            pl.BlockSpec((1, C, H * W), lambda i: (i, 0, 0)),
        ],
        out_shape=[
            jax.ShapeDtypeStruct((B, 1, H * W), jnp.int32),
            jax.ShapeDtypeStruct((B, C, H * W), jnp.float32),
        ],
    )(zf, embedding_weight)
    return idx_out.reshape(B, 1, H, W), zq_out.reshape(B, C, H, W)
